# Initial kernel scaffold; baseline (speedup 1.0000x reference)
import jax
import jax.numpy as jnp
from jax.experimental import pallas as pl


def kernel(x, edge_attr, params, edge_index):
    raise NotImplementedError("write your pallas kernel here")


# speedup vs baseline: 1.1015x; 1.1015x over previous
import functools

import jax
import jax.numpy as jnp
from jax import lax
from jax.experimental import pallas as pl
from jax.experimental.pallas import tpu as pltpu
from jax.experimental.pallas import tpu_sc as plsc

N_NODES = 50000
N_EDGES = 800000
LATENT = 64

_NC = 2
_NS = 16
_NW = _NC * _NS

_EP = 819200

_GW = 128
_GROWS = _EP // _GW
_GROWS_W = _GROWS // _NW
_GCH = 8
_GH = 2
_TW = 2 * LATENT

_SCH = 16
_SSUB = 1
_NBUF = 3
_SROWS_T = _GROWS // _NS
_SITERS = _SROWS_T // _SCH
_Q = N_NODES // 4
_TOUT = 784
_ACC_ROWS = _NS * _TOUT


def _mesh():
  return plsc.VectorSubcoreMesh(core_axis_name="c", subcore_axis_name="s",
                                num_cores=_NC, num_subcores=_NS)


def _gather2(table, dsti, srci):
  @functools.partial(
      pl.kernel,
      out_type=(jax.ShapeDtypeStruct((_EP, _TW), jnp.float32),
                jax.ShapeDtypeStruct((_EP, _TW), jnp.float32)),
      mesh=_mesh(),
      scratch_types=[
          pltpu.VMEM((_GCH, _GW), jnp.int32),
          pltpu.VMEM((_GCH, _GW), jnp.int32),
          pltpu.VMEM((_GH * _GW, _TW), jnp.float32),
          pltpu.VMEM((_GH * _GW, _TW), jnp.float32),
          pltpu.SemaphoreType.DMA,
          pltpu.SemaphoreType.DMA,
      ],
  )
  def k(table_h, dsti_h, srci_h, outd_h, outs_h, idxd, idxs, rowsd, rowss,
        semd, sems):
    wid = lax.axis_index("s") * _NC + lax.axis_index("c")
    row0 = wid * _GROWS_W

    def body(i, carry):
      r = row0 + i * _GCH
      pltpu.sync_copy(dsti_h.at[pl.ds(r, _GCH)], idxd)
      pltpu.sync_copy(srci_h.at[pl.ds(r, _GCH)], idxs)
      for h in range(_GCH // _GH):
        cps = []
        for j in range(_GH):
          row = h * _GH + j
          cps.append(pltpu.async_copy(
              table_h.at[idxd.at[row]], rowsd.at[pl.ds(j * _GW, _GW)], semd))
          cps.append(pltpu.async_copy(
              table_h.at[idxs.at[row]], rowss.at[pl.ds(j * _GW, _GW)], sems))
        for cp in cps:
          cp.wait()
        e0 = (r + h * _GH) * _GW
        pltpu.sync_copy(rowsd, outd_h.at[pl.ds(e0, _GH * _GW)])
        pltpu.sync_copy(rowss, outs_h.at[pl.ds(e0, _GH * _GW)])
      return carry

    lax.fori_loop(0, _GROWS_W // _GCH, body, 0)

  return k(table, dsti, srci)


def _scatter_sum(rows, dsti, zrows, q):
  @functools.partial(
      pl.kernel,
      out_type=jax.ShapeDtypeStruct((2 * _ACC_ROWS, LATENT), jnp.float32),
      mesh=_mesh(),
      scratch_types=[
          pltpu.VMEM((_SCH, _GW), jnp.int32),
          pltpu.VMEM((_SCH, _GW), jnp.int32),
          [pltpu.VMEM((_SSUB * _GW, LATENT), jnp.float32)
           for _ in range(_NBUF)],
          pltpu.VMEM_SHARED((_ACC_ROWS, LATENT), jnp.float32),
          pltpu.SemaphoreType.DMA,
          pltpu.SemaphoreType.DMA,
      ],
  )
  def k(rows_h, dsti_h, z_h, out_h, idxr, idxm, bufs, acc, seml, sema):
    cid = lax.axis_index("c")
    sid = lax.axis_index("s")
    node0 = (2 * q + cid) * _Q
    pltpu.sync_copy(z_h, acc.at[pl.ds(sid * _TOUT, _TOUT)])
    plsc.subcore_barrier()
    nsub = _SCH // _SSUB

    def fire_adds(s, buf):
      cps = []
      for j in range(_SSUB):
        cps.append(pltpu.async_copy(
            buf.at[pl.ds(j * _GW, _GW)],
            acc.at[idxm.at[s * _SSUB + j]], sema, add=True))
      return cps

    def body(i, carry):
      r = sid * _SROWS_T + i * _SCH
      pltpu.sync_copy(dsti_h.at[pl.ds(r, _SCH)], idxr)
      for h in range(_SCH):
        for kk in range(_GW // 16):
          v = idxr[h, pl.ds(kk * 16, 16)]
          lv = v - node0
          inb = (lv >= 0) & (lv < _Q)
          idxm[h, pl.ds(kk * 16, 16)] = jnp.where(inb, lv, _Q + sid)
      loads = [None] * nsub
      adds = [None] * nsub
      for s in range(nsub):
        if s >= _NBUF:
          for cp in adds[s - _NBUF]:
            cp.wait()
        loads[s] = pltpu.async_copy(
            rows_h.at[pl.ds((r + s * _SSUB) * _GW, _SSUB * _GW)],
            bufs[s % _NBUF], seml)
        if s >= 1:
          loads[s - 1].wait()
          adds[s - 1] = fire_adds(s - 1, bufs[(s - 1) % _NBUF])
      loads[nsub - 1].wait()
      adds[nsub - 1] = fire_adds(nsub - 1, bufs[(nsub - 1) % _NBUF])
      for s in range(nsub - _NBUF, nsub):
        for cp in adds[s]:
          cp.wait()
      return carry

    lax.fori_loop(0, _SITERS, body, 0)
    plsc.subcore_barrier()
    pltpu.sync_copy(acc.at[pl.ds(sid * _TOUT, _TOUT)],
                    out_h.at[pl.ds(cid * _ACC_ROWS + sid * _TOUT, _TOUT)])

  return k(rows, dsti, zrows)


def _fnet_tc(parts, p, residual=None, block=2000, logical=None):
  m = parts[0].shape[0]
  dims = [q.shape[1] for q in parts]
  n = len(parts)
  if logical is None:
    logical = dims
  w_in = p["in"]["W"]
  ws, off = [], 0
  for dd, lg in zip(dims, logical):
    w = w_in[off:off + lg]
    if dd > lg:
      w = jnp.concatenate([w, jnp.zeros((dd - lg, w.shape[1]), w.dtype)])
    ws.append(w)
    off += lg
  rb = p["res"][0]
  has_ln = "ln" in p
  out_dim = p["out"]["W"].shape[1]
  nres = 1 if residual is not None else 0

  def body(*refs):
    part_refs = refs[:n]
    pos = n
    res_ref = refs[pos] if nres else None
    pos += nres
    w_refs = refs[pos:pos + n]
    pos += n
    b_in, wr1, br1, wr2, br2, wo, bo = refs[pos:pos + 7]
    pos += 7
    if has_ln:
      g_ref, bl_ref = refs[pos:pos + 2]
    out_ref = refs[-1]

    dot = lambda a, b: jnp.dot(a, b, preferred_element_type=jnp.float32)
    acc = dot(part_refs[0][...], w_refs[0][...]) + b_in[...]
    for t in range(1, n):
      acc = acc + dot(part_refs[t][...], w_refs[t][...])
    h = jnp.maximum(acc, 0.0)
    h2 = jnp.maximum(dot(h, wr1[...]) + br1[...], 0.0)
    h2 = jnp.maximum(dot(h2, wr2[...]) + br2[...], 0.0)
    h = h + h2
    o = dot(h, wo[...]) + bo[...]
    if has_ln:
      mu = jnp.mean(o, axis=1, keepdims=True)
      var = jnp.mean((o - mu) * (o - mu), axis=1, keepdims=True)
      o = (o - mu) * lax.rsqrt(var + 1e-5) * g_ref[...] + bl_ref[...]
    if nres:
      o = res_ref[...] + o
    out_ref[...] = o

  row2 = lambda a: a.reshape(1, -1)
  weights = list(ws) + [row2(p["in"]["b"]),
                        rb["l1"]["W"], row2(rb["l1"]["b"]),
                        rb["l2"]["W"], row2(rb["l2"]["b"]),
                        p["out"]["W"], row2(p["out"]["b"])]
  if has_ln:
    weights += [row2(p["ln"]["g"]), row2(p["ln"]["b"])]

  in_specs = [pl.BlockSpec((block, dd), lambda i: (i, 0)) for dd in dims]
  if nres:
    in_specs.append(pl.BlockSpec((block, LATENT), lambda i: (i, 0)))
  for w in weights:
    in_specs.append(pl.BlockSpec(w.shape, lambda i: (0, 0)))

  args = list(parts) + ([residual] if nres else []) + weights
  return pl.pallas_call(
      body,
      grid=(m // block,),
      in_specs=in_specs,
      out_specs=pl.BlockSpec((block, out_dim), lambda i: (i, 0)),
      out_shape=jax.ShapeDtypeStruct((m, out_dim), jnp.float32),
  )(*args)


def kernel(x, edge_attr, params, edge_index):
  src = edge_index[0]
  dst = edge_index[1]
  pad = _EP - N_EDGES
  dst_g = jnp.concatenate([dst, jnp.zeros((pad,), jnp.int32)])
  src_g = jnp.concatenate([src, jnp.zeros((pad,), jnp.int32)])
  dsti_g = dst_g.reshape(_GROWS, _GW)
  srci_g = src_g.reshape(_GROWS, _GW)
  dsti_s = jnp.concatenate([dst, jnp.full((pad,), N_NODES, jnp.int32)])
  ea_p = jnp.concatenate([edge_attr,
                          jnp.zeros((pad, edge_attr.shape[1]), jnp.float32)])

  zcols = jnp.zeros((N_NODES, LATENT), jnp.float32)

  node_latents = _fnet_tc([x], params["node_enc"])
  edge_latents = _fnet_tc([ea_p], params["edge_enc"], block=1600)
  for lp in params["proc"]:
    table = jnp.concatenate([node_latents, zcols], axis=1)
    gd, gs = _gather2(table, dsti_g, srci_g)
    new_edge = _fnet_tc([gd, gs, edge_latents], lp["edge"],
                        residual=edge_latents, block=1600,
                        logical=[LATENT, LATENT, LATENT])
    agg = jax.ops.segment_sum(new_edge, dsti_s, num_segments=N_NODES)
    node_latents = _fnet_tc([node_latents, agg], lp["node"],
                            residual=node_latents)
    edge_latents = new_edge
  return _fnet_tc([node_latents], params["dec"])
